# Initial kernel scaffold; baseline (speedup 1.0000x reference)
import jax
import jax.numpy as jnp
from jax.experimental import pallas as pl


def kernel(x, edge_index, edge_attr, We1, W1a, bn1w, bn1b, W1b, We2, W2a, bn2w, bn2b, W2b):
    raise NotImplementedError("write your pallas kernel here")


# speedup vs baseline: 2.2033x; 2.2033x over previous
import functools

import jax
import jax.numpy as jnp
from jax import lax
from jax.experimental import pallas as pl
from jax.experimental.pallas import tpu as pltpu
from jax.experimental.pallas import tpu_sc as plsc

N = 10000
E = 320000
D = 128
DE = 16
H = 256
EPS = 1e-7
BN_EPS = 1e-5

DH = D // 2
C = 128
NTILES = 16
NCHUNKS = 157
EPT = NCHUNKS * C
E_PAD = NTILES * EPT
NP = 10112
RPT = NP // NTILES
BE = 512
NEB = E_PAD // BE
BN = 1000


def _edge_mm_body(ea_ref, we_ref, o_ref):
    o_ref[...] = jnp.dot(ea_ref[...], we_ref[...],
                         preferred_element_type=jnp.float32)


def _edge_mm(ea_pad, We):
    return pl.pallas_call(
        _edge_mm_body,
        grid=(NEB,),
        in_specs=[
            pl.BlockSpec((BE, DE), lambda i: (i, 0)),
            pl.BlockSpec((DE, D), lambda i: (0, 0)),
        ],
        out_specs=pl.BlockSpec((BE, D), lambda i: (i, 0)),
        out_shape=jax.ShapeDtypeStruct((E_PAD, D), jnp.float32),
    )(ea_pad, We)


_sc_mesh = plsc.VectorSubcoreMesh(core_axis_name="c", subcore_axis_name="s",
                                  num_cores=2, num_subcores=16)


@functools.partial(
    pl.kernel,
    out_type=jax.ShapeDtypeStruct((2 * NP, D), jnp.float32),
    mesh=_sc_mesh,
    scratch_types=[
        pltpu.VMEM((C,), jnp.int32),
        pltpu.VMEM((C,), jnp.int32),
        pltpu.VMEM((C, D), jnp.float32),
        pltpu.VMEM((C, D), jnp.float32),
        pltpu.VMEM((C, D), jnp.float32),
        pltpu.VMEM_SHARED((NP, D), jnp.float32),
        pltpu.SemaphoreType.DMA,
    ],
)
def _sc_agg(x_h, e_h, src_h, dst_h, out2,
            srcv, dstv, gbuf, ebuf, obuf, acc, gsem):
    c = lax.axis_index("c")
    s = lax.axis_index("s")
    zero = jnp.zeros((16,), jnp.float32)

    def zrow(i, carry):
        for j in range(D // 16):
            obuf[i, pl.ds(j * 16, 16)] = zero
        return carry

    lax.fori_loop(0, C, zrow, 0)
    for k in range(4):
        pltpu.sync_copy(obuf, acc.at[pl.ds(s * RPT + k * 128, 128)])
    pltpu.sync_copy(obuf.at[pl.ds(0, RPT - 512)],
                    acc.at[pl.ds(s * RPT + 512, RPT - 512)])
    plsc.subcore_barrier()

    base0 = s * EPT
    ch = pl.multiple_of(c * DH, DH)

    def chunk(i, carry):
        base = base0 + i * C
        pltpu.sync_copy(src_h.at[pl.ds(base, C)], srcv)
        pltpu.sync_copy(dst_h.at[pl.ds(base, C)], dstv)
        gcp = pltpu.async_copy(x_h.at[srcv], gbuf, gsem)
        pltpu.sync_copy(e_h.at[pl.ds(base, C)], ebuf)
        gcp.wait()

        def row(r, cc):
            for j in range(DH // 16):
                sl = pl.ds(ch + j * 16, 16)
                m = jnp.maximum(gbuf[r, sl] + ebuf[r, sl], 0.0) + EPS
                t = jnp.exp(m)
                obuf[r, pl.ds(j * 16, 16)] = t
                obuf[r, pl.ds(DH + j * 16, 16)] = m * t
            return cc

        lax.fori_loop(0, C, row, 0)
        pltpu.sync_copy(obuf, acc.at[dstv], add=True)
        return carry

    lax.fori_loop(0, NCHUNKS, chunk, 0)
    plsc.subcore_barrier()

    for k in range(4):
        off = s * RPT + k * 128
        pltpu.sync_copy(acc.at[pl.ds(off, 128)], obuf)
        pltpu.sync_copy(obuf, out2.at[pl.ds(c * NP + off, 128)])
    off = s * RPT + 512
    vb = obuf.at[pl.ds(0, RPT - 512)]
    pltpu.sync_copy(acc.at[pl.ds(off, RPT - 512)], vb)
    pltpu.sync_copy(vb, out2.at[pl.ds(c * NP + off, RPT - 512)])


def _node_mlp_body(relu_out, num_ref, den_ref, x_ref, wa_ref, s1_ref, b1_ref,
                   wb_ref, o_ref):
    den = den_ref[...]
    agg = num_ref[...] / jnp.where(den == 0.0, 1.0, den)
    o = agg + x_ref[...]
    h = jnp.dot(o, wa_ref[...], preferred_element_type=jnp.float32)
    h = jnp.maximum(h * s1_ref[...] + b1_ref[...], 0.0)
    y = jnp.dot(h, wb_ref[...], preferred_element_type=jnp.float32)
    if relu_out:
        y = jnp.maximum(y, 0.0)
    o_ref[...] = y


def _node_mlp(num, den, x, Wa, s1, b1, Wb, relu_out):
    return pl.pallas_call(
        functools.partial(_node_mlp_body, relu_out),
        grid=(N // BN,),
        in_specs=[
            pl.BlockSpec((BN, D), lambda i: (i, 0)),
            pl.BlockSpec((BN, D), lambda i: (i, 0)),
            pl.BlockSpec((BN, D), lambda i: (i, 0)),
            pl.BlockSpec((D, H), lambda i: (0, 0)),
            pl.BlockSpec((1, H), lambda i: (0, 0)),
            pl.BlockSpec((1, H), lambda i: (0, 0)),
            pl.BlockSpec((H, D), lambda i: (0, 0)),
        ],
        out_specs=pl.BlockSpec((BN, D), lambda i: (i, 0)),
        out_shape=jax.ShapeDtypeStruct((N, D), jnp.float32),
    )(num, den, x, Wa, s1, b1, Wb)


def _layer(xin, src, dst, ea_pad, We, Wa, bnw, bnb, Wb, relu_out):
    e_rows = _edge_mm(ea_pad, We)
    out2 = _sc_agg(xin, e_rows, src, dst)
    den = jnp.concatenate([out2[:N, :DH], out2[NP:NP + N, :DH]], axis=1)
    num = jnp.concatenate([out2[:N, DH:], out2[NP:NP + N, DH:]], axis=1)
    s1 = (bnw / jnp.sqrt(1.0 + BN_EPS)).reshape(1, H)
    b1 = bnb.reshape(1, H)
    return _node_mlp(num, den, xin, Wa, s1, b1, Wb, relu_out)


def kernel(x, edge_index, edge_attr, We1, W1a, bn1w, bn1b, W1b,
           We2, W2a, bn2w, bn2b, W2b):
    pad = E_PAD - E
    src = jnp.concatenate([edge_index[0], jnp.zeros((pad,), jnp.int32)])
    dst = jnp.concatenate([edge_index[1], jnp.full((pad,), N, jnp.int32)])
    ea_pad = jnp.concatenate(
        [edge_attr, jnp.zeros((pad, DE), jnp.float32)], axis=0)
    h = _layer(x, src, dst, ea_pad, We1, W1a, bn1w, bn1b, W1b, True)
    return _layer(h, src, dst, ea_pad, We2, W2a, bn2w, bn2b, W2b, False)


# speedup vs baseline: 3.7097x; 1.6837x over previous
import functools

import jax
import jax.numpy as jnp
from jax import lax
from jax.experimental import pallas as pl
from jax.experimental.pallas import tpu as pltpu
from jax.experimental.pallas import tpu_sc as plsc

N = 10000
E = 320000
D = 128
DE = 16
H = 256
EPS = 1e-7
BN_EPS = 1e-5

DH = D // 2
C = 64
NTILES = 16
CPS = 64
NSUP = 5
EPT = NSUP * CPS * C
E_PAD = NTILES * EPT
NP = 10112
RPT = NP // NTILES
BE = 512
NEB = E_PAD // BE
BN = 1000


def _edge_mm_body(ea_ref, we_ref, o_ref):
    o_ref[...] = jnp.dot(ea_ref[...], we_ref[...],
                         preferred_element_type=jnp.float32)


def _edge_mm(ea_pad, We):
    return pl.pallas_call(
        _edge_mm_body,
        grid=(NEB,),
        in_specs=[
            pl.BlockSpec((BE, DE), lambda i: (i, 0)),
            pl.BlockSpec((DE, D), lambda i: (0, 0)),
        ],
        out_specs=pl.BlockSpec((BE, D), lambda i: (i, 0)),
        out_shape=jax.ShapeDtypeStruct((E_PAD, D), jnp.float32),
    )(ea_pad, We)


_sc_mesh = plsc.VectorSubcoreMesh(core_axis_name="c", subcore_axis_name="s",
                                  num_cores=2, num_subcores=16)


@functools.partial(
    pl.kernel,
    out_type=jax.ShapeDtypeStruct((2 * NP, D), jnp.float32),
    mesh=_sc_mesh,
    scratch_types=[
        pltpu.VMEM((CPS, C), jnp.int32),
        pltpu.VMEM((2, C), jnp.int32),
        pltpu.VMEM((2, C, D), jnp.float32),
        pltpu.VMEM((2, C, D), jnp.float32),
        pltpu.VMEM((C, D), jnp.float32),
        pltpu.VMEM_SHARED((NP, D), jnp.float32),
        pltpu.SemaphoreType.DMA,
        pltpu.SemaphoreType.DMA,
        pltpu.SemaphoreType.DMA,
        pltpu.SemaphoreType.DMA,
        pltpu.SemaphoreType.DMA,
        pltpu.SemaphoreType.DMA,
    ],
)
def _sc_agg(x_h, e_h, src2_h, dst_h, out2,
            srcsup, dstv, gbuf, ebuf, obuf, acc, sg0, sg1, se0, se1, sd0, sd1):
    c = lax.axis_index("c")
    s = lax.axis_index("s")
    zero = jnp.zeros((16,), jnp.float32)
    sg = (sg0, sg1)
    se = (se0, se1)
    sd = (sd0, sd1)

    def zrow(i, carry):
        for j in range(D // 16):
            obuf[i, pl.ds(j * 16, 16)] = zero
        return carry

    lax.fori_loop(0, C, zrow, 0)
    for k in range(9):
        pltpu.sync_copy(obuf, acc.at[pl.ds(s * RPT + k * C, C)])
    pltpu.sync_copy(obuf.at[pl.ds(0, RPT - 576)],
                    acc.at[pl.ds(s * RPT + 576, RPT - 576)])
    plsc.subcore_barrier()

    ch = pl.multiple_of(c * DH, DH)
    ebase0 = s * EPT
    rbase0 = s * (NSUP * CPS)

    def issue(k, b):
        geb = ebase0_t + k * C
        pltpu.async_copy(dst_h.at[pl.ds(geb, C)], dstv.at[b], sd[b])
        pltpu.async_copy(x_h.at[srcsup.at[k]], gbuf.at[b], sg[b])
        pltpu.async_copy(e_h.at[pl.ds(geb, C)], ebuf.at[b], se[b])

    def waitfor(k, b):
        geb = ebase0_t + k * C
        pltpu.make_async_copy(dst_h.at[pl.ds(geb, C)], dstv.at[b], sd[b]).wait()
        pltpu.make_async_copy(x_h.at[srcsup.at[k]], gbuf.at[b], sg[b]).wait()
        pltpu.make_async_copy(e_h.at[pl.ds(geb, C)], ebuf.at[b], se[b]).wait()

    def do_chunk(k, b):
        @plsc.parallel_loop(0, C, step=1, unroll=4)
        def rowfn(r):
            for j in range(DH // 16):
                sl = pl.ds(ch + j * 16, 16)
                m = jnp.maximum(gbuf[b, r, sl] + ebuf[b, r, sl], 0.0) + EPS
                t = jnp.exp(m)
                obuf[r, pl.ds(j * 16, 16)] = t
                obuf[r, pl.ds(DH + j * 16, 16)] = m * t

        pltpu.sync_copy(obuf, acc.at[dstv.at[b]], add=True)

    for t in range(NSUP):
        ebase0_t = ebase0 + t * CPS * C
        pltpu.sync_copy(src2_h.at[pl.ds(rbase0 + t * CPS, CPS)], srcsup)
        issue(0, 0)

        def pair(p, carry):
            k0 = 2 * p
            issue(k0 + 1, 1)
            waitfor(k0, 0)
            do_chunk(k0, 0)
            k1 = 2 * p + 1
            knext = jnp.minimum(k1 + 1, CPS - 1)
            issue(knext, 0)
            waitfor(k1, 1)
            do_chunk(k1, 1)
            return carry

        lax.fori_loop(0, CPS // 2, pair, 0)
        waitfor(CPS - 1, 0)

    plsc.subcore_barrier()

    for k in range(9):
        off = s * RPT + k * C
        pltpu.sync_copy(acc.at[pl.ds(off, C)], obuf)
        pltpu.sync_copy(obuf, out2.at[pl.ds(c * NP + off, C)])
    off = s * RPT + 576
    vb = obuf.at[pl.ds(0, RPT - 576)]
    pltpu.sync_copy(acc.at[pl.ds(off, RPT - 576)], vb)
    pltpu.sync_copy(vb, out2.at[pl.ds(c * NP + off, RPT - 576)])


def _node_mlp_body(relu_out, num_ref, den_ref, x_ref, wa_ref, s1_ref, b1_ref,
                   wb_ref, o_ref):
    den = den_ref[...]
    agg = num_ref[...] / jnp.where(den == 0.0, 1.0, den)
    o = agg + x_ref[...]
    h = jnp.dot(o, wa_ref[...], preferred_element_type=jnp.float32)
    h = jnp.maximum(h * s1_ref[...] + b1_ref[...], 0.0)
    y = jnp.dot(h, wb_ref[...], preferred_element_type=jnp.float32)
    if relu_out:
        y = jnp.maximum(y, 0.0)
    o_ref[...] = y


def _node_mlp(num, den, x, Wa, s1, b1, Wb, relu_out):
    return pl.pallas_call(
        functools.partial(_node_mlp_body, relu_out),
        grid=(N // BN,),
        in_specs=[
            pl.BlockSpec((BN, D), lambda i: (i, 0)),
            pl.BlockSpec((BN, D), lambda i: (i, 0)),
            pl.BlockSpec((BN, D), lambda i: (i, 0)),
            pl.BlockSpec((D, H), lambda i: (0, 0)),
            pl.BlockSpec((1, H), lambda i: (0, 0)),
            pl.BlockSpec((1, H), lambda i: (0, 0)),
            pl.BlockSpec((H, D), lambda i: (0, 0)),
        ],
        out_specs=pl.BlockSpec((BN, D), lambda i: (i, 0)),
        out_shape=jax.ShapeDtypeStruct((N, D), jnp.float32),
    )(num, den, x, Wa, s1, b1, Wb)


def _layer(xin, src2, dst, ea_pad, We, Wa, bnw, bnb, Wb, relu_out):
    e_rows = _edge_mm(ea_pad, We)
    out2 = _sc_agg(xin, e_rows, src2, dst)
    den = jnp.concatenate([out2[:N, :DH], out2[NP:NP + N, :DH]], axis=1)
    num = jnp.concatenate([out2[:N, DH:], out2[NP:NP + N, DH:]], axis=1)
    s1 = (bnw / jnp.sqrt(1.0 + BN_EPS)).reshape(1, H)
    b1 = bnb.reshape(1, H)
    return _node_mlp(num, den, xin, Wa, s1, b1, Wb, relu_out)


def kernel(x, edge_index, edge_attr, We1, W1a, bn1w, bn1b, W1b,
           We2, W2a, bn2w, bn2b, W2b):
    pad = E_PAD - E
    src2 = jnp.concatenate(
        [edge_index[0], jnp.zeros((pad,), jnp.int32)]).reshape(E_PAD // C, C)
    dst = jnp.concatenate([edge_index[1], jnp.full((pad,), N, jnp.int32)])
    ea_pad = jnp.concatenate(
        [edge_attr, jnp.zeros((pad, DE), jnp.float32)], axis=0)
    h = _layer(x, src2, dst, ea_pad, We1, W1a, bn1w, bn1b, W1b, True)
    return _layer(h, src2, dst, ea_pad, We2, W2a, bn2w, bn2b, W2b, False)
